# Initial kernel scaffold; baseline (speedup 1.0000x reference)
import functools

import jax
import jax.numpy as jnp
from jax import lax
from jax.experimental import pallas as pl
from jax.experimental.pallas import tpu as pltpu
from jax.experimental.pallas import tpu_sc as plsc

VOCAB = 100000
EMBED = 64
HIDDEN = 128
BATCH = 1024
SEQ = 50

NUM_WORKERS = 32
ROWS_PER_W = BATCH * SEQ // NUM_WORKERS
CHUNK = 80
NCHUNK = ROWS_PER_W // CHUNK


def _gather_body(table_hbm, idx_hbm, out_hbm, idx_v, rows_v, sem):
    c = lax.axis_index("c")
    s = lax.axis_index("s")
    wid = s * 2 + c
    base = wid * ROWS_PER_W
    pltpu.sync_copy(idx_hbm.at[wid], idx_v)
    copies = []
    for j in range(NCHUNK):
        copies.append(
            pltpu.async_copy(
                table_hbm.at[idx_v.at[j]],
                rows_v.at[pl.ds(j * CHUNK, CHUNK)],
                sem,
            )
        )
    for cp in copies:
        cp.wait()
    pltpu.sync_copy(rows_v, out_hbm.at[pl.ds(base, ROWS_PER_W)])


def _sc_gather(word_emb, idx3):
    mesh = plsc.VectorSubcoreMesh(core_axis_name="c", subcore_axis_name="s")
    run = pl.kernel(
        _gather_body,
        mesh=mesh,
        out_type=jax.ShapeDtypeStruct((SEQ * BATCH, EMBED), jnp.float32),
        scratch_types=[
            pltpu.VMEM((NCHUNK, CHUNK), jnp.int32),
            pltpu.VMEM((ROWS_PER_W, EMBED), jnp.float32),
            pltpu.SemaphoreType.DMA,
        ],
    )
    return run(word_emb, idx3)


def _lstm_body(len_ref, xs_ref, wx_ref, wh_ref, b_ref, h_out, c_scr):
    t = pl.program_id(0)

    @pl.when(t == 0)
    def _init():
        h_out[...] = jnp.zeros_like(h_out)
        c_scr[...] = jnp.zeros_like(c_scr)

    h = h_out[...]
    c = c_scr[...]
    x = xs_ref[0]
    gates = (
        jnp.dot(x, wx_ref[...], preferred_element_type=jnp.float32)
        + jnp.dot(h, wh_ref[...], preferred_element_type=jnp.float32)
        + b_ref[...]
    )
    i_g = jax.nn.sigmoid(gates[:, 0 * HIDDEN:1 * HIDDEN])
    f_g = jax.nn.sigmoid(gates[:, 1 * HIDDEN:2 * HIDDEN])
    g_g = jnp.tanh(gates[:, 2 * HIDDEN:3 * HIDDEN])
    o_g = jax.nn.sigmoid(gates[:, 3 * HIDDEN:4 * HIDDEN])
    c_new = f_g * c + i_g * g_g
    h_new = o_g * jnp.tanh(c_new)
    m = len_ref[...] > t
    h_out[...] = jnp.where(m, h_new, h)
    c_scr[...] = jnp.where(m, c_new, c)


def _tc_lstm(len2, xs, wx, wh, bias, interpret=False):
    return pl.pallas_call(
        _lstm_body,
        grid=(SEQ,),
        in_specs=[
            pl.BlockSpec((BATCH, 1), lambda t: (0, 0)),
            pl.BlockSpec((1, BATCH, EMBED), lambda t: (t, 0, 0)),
            pl.BlockSpec((EMBED, 4 * HIDDEN), lambda t: (0, 0)),
            pl.BlockSpec((HIDDEN, 4 * HIDDEN), lambda t: (0, 0)),
            pl.BlockSpec((1, 4 * HIDDEN), lambda t: (0, 0)),
        ],
        out_specs=pl.BlockSpec((BATCH, HIDDEN), lambda t: (0, 0)),
        out_shape=jax.ShapeDtypeStruct((BATCH, HIDDEN), jnp.float32),
        scratch_shapes=[pltpu.VMEM((BATCH, HIDDEN), jnp.float32)],
        interpret=interpret,
    )(len2, xs, wx, wh, bias)


def kernel(left, left_len, word_emb, W_ih, W_hh, b_ih, b_hh):
    idx3 = jnp.transpose(left).reshape(NUM_WORKERS, NCHUNK, CHUNK).astype(jnp.int32)
    emb_flat = _sc_gather(word_emb, idx3)
    xs = emb_flat.reshape(SEQ, BATCH, EMBED)
    wx = W_ih.T
    wh = W_hh.T
    bias = (b_ih + b_hh).reshape(1, 4 * HIDDEN)
    len2 = left_len.reshape(BATCH, 1).astype(jnp.int32)
    return _tc_lstm(len2, xs, wx, wh, bias)


# speedup vs baseline: 2.7605x; 2.7605x over previous
import functools

import jax
import jax.numpy as jnp
from jax import lax
from jax.experimental import pallas as pl
from jax.experimental.pallas import tpu as pltpu
from jax.experimental.pallas import tpu_sc as plsc

VOCAB = 100000
EMBED = 64
HIDDEN = 128
BATCH = 1024
SEQ = 50

NUM_WORKERS = 32
ROWS_PER_W = BATCH * SEQ // NUM_WORKERS
CHUNK = 80
NCHUNK = ROWS_PER_W // CHUNK


def _gather_body(table_hbm, idx_hbm, out_hbm, idx_v, rows_v, sem):
    c = lax.axis_index("c")
    s = lax.axis_index("s")
    wid = s * 2 + c
    base = wid * ROWS_PER_W
    pltpu.sync_copy(idx_hbm.at[wid], idx_v)
    copies = []
    for j in range(NCHUNK):
        copies.append(
            pltpu.async_copy(
                table_hbm.at[idx_v.at[j]],
                rows_v.at[pl.ds(j * CHUNK, CHUNK)],
                sem,
            )
        )
    for cp in copies:
        cp.wait()
    pltpu.sync_copy(rows_v, out_hbm.at[pl.ds(base, ROWS_PER_W)])


def _sc_gather(word_emb, idx3):
    mesh = plsc.VectorSubcoreMesh(core_axis_name="c", subcore_axis_name="s")
    run = pl.kernel(
        _gather_body,
        mesh=mesh,
        out_type=jax.ShapeDtypeStruct((SEQ * BATCH, EMBED), jnp.float32),
        scratch_types=[
            pltpu.VMEM((NCHUNK, CHUNK), jnp.int32),
            pltpu.VMEM((ROWS_PER_W, EMBED), jnp.float32),
            pltpu.SemaphoreType.DMA,
        ],
        compiler_params=pltpu.CompilerParams(use_tc_tiling_on_sc=False),
    )
    return run(word_emb, idx3)


def _lstm_body(len_ref, xs_ref, wx_ref, wh_ref, b_ref, h_out, c_scr):
    t = pl.program_id(0)

    @pl.when(t == 0)
    def _init():
        h_out[...] = jnp.zeros_like(h_out)
        c_scr[...] = jnp.zeros_like(c_scr)

    h = h_out[...]
    c = c_scr[...]
    x = xs_ref[0]
    gates = (
        jnp.dot(x, wx_ref[...], preferred_element_type=jnp.float32)
        + jnp.dot(h, wh_ref[...], preferred_element_type=jnp.float32)
        + b_ref[...]
    )
    i_g = jax.nn.sigmoid(gates[:, 0 * HIDDEN:1 * HIDDEN])
    f_g = jax.nn.sigmoid(gates[:, 1 * HIDDEN:2 * HIDDEN])
    g_g = jnp.tanh(gates[:, 2 * HIDDEN:3 * HIDDEN])
    o_g = jax.nn.sigmoid(gates[:, 3 * HIDDEN:4 * HIDDEN])
    c_new = f_g * c + i_g * g_g
    h_new = o_g * jnp.tanh(c_new)
    m = len_ref[...] > t
    h_out[...] = jnp.where(m, h_new, h)
    c_scr[...] = jnp.where(m, c_new, c)


def _tc_lstm(len2, xs, wx, wh, bias, interpret=False):
    return pl.pallas_call(
        _lstm_body,
        grid=(SEQ,),
        in_specs=[
            pl.BlockSpec((BATCH, 1), lambda t: (0, 0)),
            pl.BlockSpec((1, BATCH, EMBED), lambda t: (t, 0, 0)),
            pl.BlockSpec((EMBED, 4 * HIDDEN), lambda t: (0, 0)),
            pl.BlockSpec((HIDDEN, 4 * HIDDEN), lambda t: (0, 0)),
            pl.BlockSpec((1, 4 * HIDDEN), lambda t: (0, 0)),
        ],
        out_specs=pl.BlockSpec((BATCH, HIDDEN), lambda t: (0, 0)),
        out_shape=jax.ShapeDtypeStruct((BATCH, HIDDEN), jnp.float32),
        scratch_shapes=[pltpu.VMEM((BATCH, HIDDEN), jnp.float32)],
        interpret=interpret,
    )(len2, xs, wx, wh, bias)


def kernel(left, left_len, word_emb, W_ih, W_hh, b_ih, b_hh):
    idx3 = jnp.transpose(left).reshape(NUM_WORKERS, NCHUNK, CHUNK).astype(jnp.int32)
    emb_flat = _sc_gather(word_emb, idx3)
    xs = emb_flat.reshape(SEQ, BATCH, EMBED)
    wx = W_ih.T
    wh = W_hh.T
    bias = (b_ih + b_hh).reshape(1, 4 * HIDDEN)
    len2 = left_len.reshape(BATCH, 1).astype(jnp.int32)
    return _tc_lstm(len2, xs, wx, wh, bias)


# speedup vs baseline: 2.7952x; 1.0126x over previous
import functools

import jax
import jax.numpy as jnp
from jax import lax
from jax.experimental import pallas as pl
from jax.experimental.pallas import tpu as pltpu
from jax.experimental.pallas import tpu_sc as plsc

VOCAB = 100000
EMBED = 64
HIDDEN = 128
BATCH = 1024
SEQ = 50

NUM_WORKERS = 32
ROWS_PER_W = BATCH * SEQ // NUM_WORKERS
CHUNK = 80
NCHUNK = ROWS_PER_W // CHUNK


def _gather_body(table_hbm, idx_hbm, out_hbm, idx_v, rows_v, sem):
    c = lax.axis_index("c")
    s = lax.axis_index("s")
    wid = s * 2 + c
    base = wid * ROWS_PER_W
    pltpu.sync_copy(idx_hbm.at[wid], idx_v)
    copies = []
    for j in range(NCHUNK):
        copies.append(
            pltpu.async_copy(
                table_hbm.at[idx_v.at[j]],
                rows_v.at[pl.ds(j * CHUNK, CHUNK)],
                sem,
            )
        )
    for cp in copies:
        cp.wait()
    pltpu.sync_copy(rows_v, out_hbm.at[pl.ds(base, ROWS_PER_W)])


def _sc_gather(word_emb, idx3):
    mesh = plsc.VectorSubcoreMesh(core_axis_name="c", subcore_axis_name="s")
    run = pl.kernel(
        _gather_body,
        mesh=mesh,
        out_type=jax.ShapeDtypeStruct((SEQ * BATCH, EMBED), jnp.float32),
        scratch_types=[
            pltpu.VMEM((NCHUNK, CHUNK), jnp.int32),
            pltpu.VMEM((ROWS_PER_W, EMBED), jnp.float32),
            pltpu.SemaphoreType.DMA,
        ],
        compiler_params=pltpu.CompilerParams(use_tc_tiling_on_sc=False),
    )
    return run(word_emb, idx3)


def _lstm_body(len_ref, xs_ref, wx_ref, wh_ref, b_ref, out_ref, h_scr, c_scr):
    h_scr[...] = jnp.zeros_like(h_scr)
    c_scr[...] = jnp.zeros_like(c_scr)

    def step(t, carry):
        h = h_scr[...]
        c = c_scr[...]
        x = xs_ref[t]
        gates = (
            jnp.dot(x, wx_ref[...], preferred_element_type=jnp.float32)
            + jnp.dot(h, wh_ref[...], preferred_element_type=jnp.float32)
            + b_ref[...]
        )
        tg = jnp.tanh(gates)
        i_g = tg[:, 0 * HIDDEN:1 * HIDDEN] * 0.5 + 0.5
        f_g = tg[:, 1 * HIDDEN:2 * HIDDEN] * 0.5 + 0.5
        g_g = tg[:, 2 * HIDDEN:3 * HIDDEN]
        o_g = tg[:, 3 * HIDDEN:4 * HIDDEN] * 0.5 + 0.5
        c_new = f_g * c + i_g * g_g
        h_new = o_g * jnp.tanh(c_new)
        m = len_ref[...] > t
        h_scr[...] = jnp.where(m, h_new, h)
        c_scr[...] = jnp.where(m, c_new, c)
        return carry

    lax.fori_loop(0, SEQ, step, 0)
    out_ref[...] = h_scr[...]


def _tc_lstm(len2, xs, wx, wh, bias, interpret=False):
    return pl.pallas_call(
        _lstm_body,
        out_shape=jax.ShapeDtypeStruct((BATCH, HIDDEN), jnp.float32),
        scratch_shapes=[
            pltpu.VMEM((BATCH, HIDDEN), jnp.float32),
            pltpu.VMEM((BATCH, HIDDEN), jnp.float32),
        ],
        interpret=interpret,
    )(len2, xs, wx, wh, bias)


def kernel(left, left_len, word_emb, W_ih, W_hh, b_ih, b_hh):
    idx3 = jnp.transpose(left).reshape(NUM_WORKERS, NCHUNK, CHUNK).astype(jnp.int32)
    emb_flat = _sc_gather(word_emb, idx3)
    xs = emb_flat.reshape(SEQ, BATCH, EMBED)
    scale = jnp.concatenate(
        [
            jnp.full((2 * HIDDEN,), 0.5, jnp.float32),
            jnp.ones((HIDDEN,), jnp.float32),
            jnp.full((HIDDEN,), 0.5, jnp.float32),
        ]
    )
    wx = W_ih.T * scale[None, :]
    wh = W_hh.T * scale[None, :]
    bias = ((b_ih + b_hh) * scale).reshape(1, 4 * HIDDEN)
    len2 = left_len.reshape(BATCH, 1).astype(jnp.int32)
    return _tc_lstm(len2, xs, wx, wh, bias)


# speedup vs baseline: 4.4854x; 1.6047x over previous
import jax
import jax.numpy as jnp
from jax import lax
from jax.experimental import pallas as pl
from jax.experimental.pallas import tpu as pltpu
from jax.experimental.pallas import tpu_sc as plsc

VOCAB = 100000
EMBED = 64
HIDDEN = 128
BATCH = 1024
SEQ = 50

NUM_WORKERS = 32
ROWS_PER_W = BATCH * SEQ // NUM_WORKERS
CHUNK = 80
NCHUNK = ROWS_PER_W // CHUNK

VHALF = 50048
TW = 2176
TGRID = VHALF // TW


def _transpose_body(lo_ref, hi_ref, out_ref):
    out_ref[...] = jnp.concatenate([lo_ref[...].T, hi_ref[...].T], axis=1)


def _tc_transpose(view, interpret=False):
    return pl.pallas_call(
        _transpose_body,
        grid=(TGRID,),
        in_specs=[
            pl.BlockSpec((EMBED, TW), lambda j: (0, j)),
            pl.BlockSpec((EMBED, TW), lambda j: (0, j + TGRID)),
        ],
        out_specs=pl.BlockSpec((TW, 2 * EMBED), lambda j: (j, 0)),
        out_shape=jax.ShapeDtypeStruct((VHALF, 2 * EMBED), jnp.float32),
        interpret=interpret,
    )(view, view)


def _idx_body(left_hbm, idx_hbm, left_v, idx_v):
    c = lax.axis_index("c")
    s = lax.axis_index("s")
    wid = s * 2 + c
    base = wid * ROWS_PER_W
    pltpu.sync_copy(left_hbm, left_v)
    for ch in range(NCHUNK):
        for q in range(CHUNK // 16):
            j = base + ch * CHUNK + q * 16 + lax.iota(jnp.int32, 16)
            k = j >> 1
            b = k & (BATCH - 1)
            u = k >> 10
            t = (u << 1) | (j & 1)
            vals = plsc.load_gather(left_v, [b, t])
            m = jnp.where(vals < VHALF, vals * 2, vals * 2 - (2 * VHALF - 1))
            idx_v[ch, pl.ds(q * 16, 16)] = m
    pltpu.sync_copy(idx_v, idx_hbm.at[wid])


def _sc_idx(left):
    mesh = plsc.VectorSubcoreMesh(core_axis_name="c", subcore_axis_name="s")
    run = pl.kernel(
        _idx_body,
        mesh=mesh,
        out_type=jax.ShapeDtypeStruct((NUM_WORKERS, NCHUNK, CHUNK), jnp.int32),
        scratch_types=[
            pltpu.VMEM((BATCH, SEQ), jnp.int32),
            pltpu.VMEM((NCHUNK, CHUNK), jnp.int32),
        ],
        compiler_params=pltpu.CompilerParams(
            use_tc_tiling_on_sc=False, needs_layout_passes=False
        ),
    )
    return run(left)


def _gather_body(table_hbm, idxs_hbm, out_hbm, idx_v, rows_v, sem):
    c = lax.axis_index("c")
    s = lax.axis_index("s")
    wid = s * 2 + c
    base = wid * ROWS_PER_W
    pltpu.sync_copy(idxs_hbm.at[wid], idx_v)
    copies = []
    for ch in range(NCHUNK):
        copies.append(
            pltpu.async_copy(
                table_hbm.at[idx_v.at[ch]],
                rows_v.at[pl.ds(ch * CHUNK, CHUNK)],
                sem,
            )
        )
    for cp in copies:
        cp.wait()
    pltpu.sync_copy(rows_v, out_hbm.at[pl.ds(base, ROWS_PER_W)])


def _sc_gather(table2, idxs):
    mesh = plsc.VectorSubcoreMesh(core_axis_name="c", subcore_axis_name="s")
    run = pl.kernel(
        _gather_body,
        mesh=mesh,
        out_type=jax.ShapeDtypeStruct((SEQ * BATCH, EMBED), jnp.float32),
        scratch_types=[
            pltpu.VMEM((NCHUNK, CHUNK), jnp.int32),
            pltpu.VMEM((ROWS_PER_W, EMBED), jnp.float32),
            pltpu.SemaphoreType.DMA,
        ],
        compiler_params=pltpu.CompilerParams(
            use_tc_tiling_on_sc=False, needs_layout_passes=False
        ),
    )
    return run(table2, idxs)


def _sc_gather_pipeline(word_emb, left, interpret=False):
    view = word_emb.T
    fused = _tc_transpose(view, interpret=interpret)
    table2 = fused.reshape(2 * VHALF, EMBED)
    idxs = _sc_idx(left)
    return _sc_gather(table2, idxs)


def _lstm_body(len_ref, xs_ref, we_ref, wo_ref, b_ref, out_ref, cat_scr, c_scr):
    cat_scr[...] = jnp.zeros_like(cat_scr)
    c_scr[...] = jnp.zeros_like(c_scr)

    def step(u, carry):
        fused = xs_ref[u]
        for p in range(2):
            cat_scr[:, p * EMBED:(p + 1) * EMBED] = (
                fused[:, p * EMBED:(p + 1) * EMBED]
            )
            h = cat_scr[:, 2 * EMBED:]
            c = c_scr[...]
            w_ref = we_ref if p == 0 else wo_ref
            gates = (
                jnp.dot(
                    cat_scr[...], w_ref[...],
                    preferred_element_type=jnp.float32,
                )
                + b_ref[...]
            )
            tg = jnp.tanh(gates)
            i_g = tg[:, 0 * HIDDEN:1 * HIDDEN] * 0.5 + 0.5
            f_g = tg[:, 1 * HIDDEN:2 * HIDDEN] * 0.5 + 0.5
            g_g = tg[:, 2 * HIDDEN:3 * HIDDEN]
            o_g = tg[:, 3 * HIDDEN:4 * HIDDEN] * 0.5 + 0.5
            c_new = f_g * c + i_g * g_g
            h_new = o_g * jnp.tanh(c_new)
            m = len_ref[...] > (2 * u + p)
            cat_scr[:, 2 * EMBED:] = jnp.where(m, h_new, h)
            c_scr[...] = jnp.where(m, c_new, c)
        return carry

    lax.fori_loop(0, SEQ // 2, step, 0)
    out_ref[...] = cat_scr[:, 2 * EMBED:]


def _tc_lstm(len2, xs, wcat_e, wcat_o, bias, interpret=False):
    return pl.pallas_call(
        _lstm_body,
        out_shape=jax.ShapeDtypeStruct((BATCH, HIDDEN), jnp.float32),
        scratch_shapes=[
            pltpu.VMEM((BATCH, 2 * EMBED + HIDDEN), jnp.float32),
            pltpu.VMEM((BATCH, HIDDEN), jnp.float32),
        ],
        interpret=interpret,
    )(len2, xs, wcat_e, wcat_o, bias)


def kernel(left, left_len, word_emb, W_ih, W_hh, b_ih, b_hh):
    emb_flat = _sc_gather_pipeline(word_emb, left.astype(jnp.int32))
    xs = emb_flat.reshape(SEQ // 2, BATCH, 2 * EMBED)
    scale = jnp.concatenate(
        [
            jnp.full((2 * HIDDEN,), 0.5, jnp.float32),
            jnp.ones((HIDDEN,), jnp.float32),
            jnp.full((HIDDEN,), 0.5, jnp.float32),
        ]
    )
    wx = W_ih.T * scale[None, :]
    wh = W_hh.T * scale[None, :]
    z = jnp.zeros((EMBED, 4 * HIDDEN), jnp.float32)
    wcat_e = jnp.concatenate([wx, z, wh])
    wcat_o = jnp.concatenate([z, wx, wh])
    bias = ((b_ih + b_hh) * scale).reshape(1, 4 * HIDDEN)
    len2 = left_len.reshape(BATCH, 1).astype(jnp.int32)
    return _tc_lstm(len2, xs, wcat_e, wcat_o, bias)


# speedup vs baseline: 4.8483x; 1.0809x over previous
import jax
import jax.numpy as jnp
from jax import lax
from jax.experimental import pallas as pl
from jax.experimental.pallas import tpu as pltpu
from jax.experimental.pallas import tpu_sc as plsc

VOCAB = 100000
EMBED = 64
HIDDEN = 128
BATCH = 1024
SEQ = 50

NUM_WORKERS = 32
ROWS_PER_W = BATCH * SEQ // NUM_WORKERS
CHUNK = 80
NCHUNK = ROWS_PER_W // CHUNK

VHALF = 50048
TW = 2944
TGRID = VHALF // TW


def _transpose_body(lo_ref, hi_ref, out_ref):
    out_ref[...] = jnp.concatenate([lo_ref[...].T, hi_ref[...].T], axis=1)


def _tc_transpose(view, interpret=False):
    return pl.pallas_call(
        _transpose_body,
        grid=(TGRID,),
        in_specs=[
            pl.BlockSpec((EMBED, TW), lambda j: (0, j)),
            pl.BlockSpec((EMBED, TW), lambda j: (0, j + TGRID)),
        ],
        out_specs=pl.BlockSpec((TW, 2 * EMBED), lambda j: (j, 0)),
        out_shape=jax.ShapeDtypeStruct((VHALF, 2 * EMBED), jnp.float32),
        interpret=interpret,
    )(view, view)


def _idx_body(leftT_hbm, idx_hbm, left_v, idx_v):
    c = lax.axis_index("c")
    s = lax.axis_index("s")
    wid = s * 2 + c
    base = wid * ROWS_PER_W
    pltpu.sync_copy(leftT_hbm, left_v)
    for ch in range(NCHUNK):
        for q in range(CHUNK // 16):
            j = base + ch * CHUNK + q * 16 + lax.iota(jnp.int32, 16)
            k = j >> 1
            b = k & (BATCH - 1)
            u = k >> 10
            t = (u << 1) | (j & 1)
            vals = plsc.load_gather(left_v, [t, b])
            m = jnp.where(vals < VHALF, vals * 2, vals * 2 - (2 * VHALF - 1))
            idx_v[ch, pl.ds(q * 16, 16)] = m
    pltpu.sync_copy(idx_v, idx_hbm.at[wid])


def _sc_idx(left):
    mesh = plsc.VectorSubcoreMesh(core_axis_name="c", subcore_axis_name="s")
    run = pl.kernel(
        _idx_body,
        mesh=mesh,
        out_type=jax.ShapeDtypeStruct((NUM_WORKERS, NCHUNK, CHUNK), jnp.int32),
        scratch_types=[
            pltpu.VMEM((SEQ, BATCH), jnp.int32),
            pltpu.VMEM((NCHUNK, CHUNK), jnp.int32),
        ],
        compiler_params=pltpu.CompilerParams(
            use_tc_tiling_on_sc=False, needs_layout_passes=False
        ),
    )
    return run(left)


def _gather_body(table_hbm, idxs_hbm, out_hbm, idx_v, rows_v, sem):
    c = lax.axis_index("c")
    s = lax.axis_index("s")
    wid = s * 2 + c
    base = wid * ROWS_PER_W
    pltpu.sync_copy(idxs_hbm.at[wid], idx_v)
    copies = []
    for ch in range(NCHUNK):
        copies.append(
            pltpu.async_copy(
                table_hbm.at[idx_v.at[ch]],
                rows_v.at[pl.ds(ch * CHUNK, CHUNK)],
                sem,
            )
        )
    for cp in copies:
        cp.wait()
    pltpu.sync_copy(rows_v, out_hbm.at[pl.ds(base, ROWS_PER_W)])


def _sc_gather(table2, idxs):
    mesh = plsc.VectorSubcoreMesh(core_axis_name="c", subcore_axis_name="s")
    run = pl.kernel(
        _gather_body,
        mesh=mesh,
        out_type=jax.ShapeDtypeStruct((SEQ * BATCH, EMBED), jnp.float32),
        scratch_types=[
            pltpu.VMEM((NCHUNK, CHUNK), jnp.int32),
            pltpu.VMEM((ROWS_PER_W, EMBED), jnp.float32),
            pltpu.SemaphoreType.DMA,
        ],
        compiler_params=pltpu.CompilerParams(
            use_tc_tiling_on_sc=False, needs_layout_passes=False
        ),
    )
    return run(table2, idxs)


def _sc_gather_pipeline(word_emb, left, interpret=False):
    view = word_emb.T
    fused = _tc_transpose(view, interpret=interpret)
    table2 = fused.reshape(2 * VHALF, EMBED)
    idxs = _sc_idx(left.T)
    return _sc_gather(table2, idxs)


def _lstm_body(len_ref, xs_ref, we_ref, wo_ref, b_ref, out_ref, cat_scr, c_scr):
    cat_scr[...] = jnp.zeros_like(cat_scr)
    c_scr[...] = jnp.zeros_like(c_scr)

    def step(u, carry):
        cat_scr[:, 0:2 * EMBED] = xs_ref[u]
        for p in range(2):
            h = cat_scr[:, 2 * EMBED:]
            c = c_scr[...]
            w_ref = we_ref if p == 0 else wo_ref
            gates = (
                jnp.dot(
                    cat_scr[...], w_ref[...],
                    preferred_element_type=jnp.float32,
                )
                + b_ref[...]
            )
            tg = jnp.tanh(gates)
            i_g = tg[:, 0 * HIDDEN:1 * HIDDEN] * 0.5 + 0.5
            f_g = tg[:, 1 * HIDDEN:2 * HIDDEN] * 0.5 + 0.5
            g_g = tg[:, 2 * HIDDEN:3 * HIDDEN]
            o_g = tg[:, 3 * HIDDEN:4 * HIDDEN] * 0.5 + 0.5
            c_new = f_g * c + i_g * g_g
            h_new = o_g * jnp.tanh(c_new)
            m = len_ref[...] > (2 * u + p)
            cat_scr[:, 2 * EMBED:] = jnp.where(m, h_new, h)
            c_scr[...] = jnp.where(m, c_new, c)
        return carry

    lax.fori_loop(0, SEQ // 2, step, 0, unroll=5)
    out_ref[...] = cat_scr[:, 2 * EMBED:]


def _tc_lstm(len2, xs, wcat_e, wcat_o, bias, interpret=False):
    return pl.pallas_call(
        _lstm_body,
        out_shape=jax.ShapeDtypeStruct((BATCH, HIDDEN), jnp.float32),
        scratch_shapes=[
            pltpu.VMEM((BATCH, 2 * EMBED + HIDDEN), jnp.float32),
            pltpu.VMEM((BATCH, HIDDEN), jnp.float32),
        ],
        interpret=interpret,
    )(len2, xs, wcat_e, wcat_o, bias)


def kernel(left, left_len, word_emb, W_ih, W_hh, b_ih, b_hh):
    emb_flat = _sc_gather_pipeline(word_emb, left.astype(jnp.int32))
    xs = emb_flat.reshape(SEQ // 2, BATCH, 2 * EMBED)
    scale = jnp.concatenate(
        [
            jnp.full((2 * HIDDEN,), 0.5, jnp.float32),
            jnp.ones((HIDDEN,), jnp.float32),
            jnp.full((HIDDEN,), 0.5, jnp.float32),
        ]
    )
    wx = W_ih.T * scale[None, :]
    wh = W_hh.T * scale[None, :]
    z = jnp.zeros((EMBED, 4 * HIDDEN), jnp.float32)
    wcat_e = jnp.concatenate([wx, z, wh])
    wcat_o = jnp.concatenate([z, wx, wh])
    bias = ((b_ih + b_hh) * scale).reshape(1, 4 * HIDDEN)
    len2 = left_len.reshape(BATCH, 1).astype(jnp.int32)
    return _tc_lstm(len2, xs, wcat_e, wcat_o, bias)


# speedup vs baseline: 4.9325x; 1.0174x over previous
import jax
import jax.numpy as jnp
from jax import lax
from jax.experimental import pallas as pl
from jax.experimental.pallas import tpu as pltpu
from jax.experimental.pallas import tpu_sc as plsc

VOCAB = 100000
EMBED = 64
HIDDEN = 128
BATCH = 1024
SEQ = 50

NUM_WORKERS = 32
ROWS_PER_W = BATCH * SEQ // NUM_WORKERS
CHUNK = 80
NCHUNK = ROWS_PER_W // CHUNK

VHALF = 50048
TW = 2944
TGRID = VHALF // TW


def _transpose_body(lo_ref, hi_ref, out_ref):
    out_ref[...] = jnp.concatenate([lo_ref[...].T, hi_ref[...].T], axis=1)


def _tc_transpose(view, interpret=False):
    return pl.pallas_call(
        _transpose_body,
        grid=(TGRID,),
        in_specs=[
            pl.BlockSpec((EMBED, TW), lambda j: (0, j)),
            pl.BlockSpec((EMBED, TW), lambda j: (0, j + TGRID)),
        ],
        out_specs=pl.BlockSpec((TW, 2 * EMBED), lambda j: (j, 0)),
        out_shape=jax.ShapeDtypeStruct((VHALF, 2 * EMBED), jnp.float32),
        interpret=interpret,
    )(view, view)


def _idx_body(leftT_hbm, idx_hbm, left_v, idx_v):
    c = lax.axis_index("c")
    s = lax.axis_index("s")
    wid = s * 2 + c
    base = wid * ROWS_PER_W
    pltpu.sync_copy(leftT_hbm, left_v)
    for ch in range(NCHUNK):
        for q in range(CHUNK // 16):
            j = base + ch * CHUNK + q * 16 + lax.iota(jnp.int32, 16)
            k = j >> 1
            b = k & (BATCH - 1)
            u = k >> 10
            t = (u << 1) | (j & 1)
            vals = plsc.load_gather(left_v, [t, b])
            m = jnp.where(vals < VHALF, vals * 2, vals * 2 - (2 * VHALF - 1))
            idx_v[ch, pl.ds(q * 16, 16)] = m
    pltpu.sync_copy(idx_v, idx_hbm.at[wid])


def _sc_idx(left):
    mesh = plsc.VectorSubcoreMesh(core_axis_name="c", subcore_axis_name="s")
    run = pl.kernel(
        _idx_body,
        mesh=mesh,
        out_type=jax.ShapeDtypeStruct((NUM_WORKERS, NCHUNK, CHUNK), jnp.int32),
        scratch_types=[
            pltpu.VMEM((SEQ, BATCH), jnp.int32),
            pltpu.VMEM((NCHUNK, CHUNK), jnp.int32),
        ],
        compiler_params=pltpu.CompilerParams(
            use_tc_tiling_on_sc=False, needs_layout_passes=False
        ),
    )
    return run(left)


def _gather_body(table_hbm, idxs_hbm, out_hbm, idx_v, rows_v, sem):
    c = lax.axis_index("c")
    s = lax.axis_index("s")
    wid = s * 2 + c
    base = wid * ROWS_PER_W
    pltpu.sync_copy(idxs_hbm.at[wid], idx_v)
    copies = []
    for ch in range(NCHUNK):
        copies.append(
            pltpu.async_copy(
                table_hbm.at[idx_v.at[ch]],
                rows_v.at[pl.ds(ch * CHUNK, CHUNK)],
                sem,
            )
        )
    for cp in copies:
        cp.wait()
    pltpu.sync_copy(rows_v, out_hbm.at[pl.ds(base, ROWS_PER_W)])


def _sc_gather(table2, idxs):
    mesh = plsc.VectorSubcoreMesh(core_axis_name="c", subcore_axis_name="s")
    run = pl.kernel(
        _gather_body,
        mesh=mesh,
        out_type=jax.ShapeDtypeStruct((SEQ * BATCH, EMBED), jnp.float32),
        scratch_types=[
            pltpu.VMEM((NCHUNK, CHUNK), jnp.int32),
            pltpu.VMEM((ROWS_PER_W, EMBED), jnp.float32),
            pltpu.SemaphoreType.DMA,
        ],
        compiler_params=pltpu.CompilerParams(
            use_tc_tiling_on_sc=False, needs_layout_passes=False
        ),
    )
    return run(table2, idxs)


def _sc_gather_pipeline(word_emb, left, interpret=False):
    view = word_emb.T
    fused = _tc_transpose(view, interpret=interpret)
    table2 = fused.reshape(2 * VHALF, EMBED)
    idxs = _sc_idx(left.T)
    return _sc_gather(table2, idxs)


def _lstm_body(len_ref, xs_ref, we_ref, wo_ref, b_ref, out_ref, cat_scr, c_scr):
    u = pl.program_id(0)

    @pl.when(u == 0)
    def _init():
        cat_scr[...] = jnp.zeros_like(cat_scr)
        c_scr[...] = jnp.zeros_like(c_scr)

    cat_scr[:, 0:2 * EMBED] = xs_ref[0]
    for p in range(2):
        h = cat_scr[:, 2 * EMBED:]
        c = c_scr[...]
        w_ref = we_ref if p == 0 else wo_ref
        gates = (
            jnp.dot(
                cat_scr[...], w_ref[...],
                preferred_element_type=jnp.float32,
            )
            + b_ref[...]
        )
        tg = jnp.tanh(gates)
        i_g = tg[:, 0 * HIDDEN:1 * HIDDEN] * 0.5 + 0.5
        f_g = tg[:, 1 * HIDDEN:2 * HIDDEN] * 0.5 + 0.5
        g_g = tg[:, 2 * HIDDEN:3 * HIDDEN]
        o_g = tg[:, 3 * HIDDEN:4 * HIDDEN] * 0.5 + 0.5
        c_new = f_g * c + i_g * g_g
        h_new = o_g * jnp.tanh(c_new)
        m = len_ref[...] > (2 * u + p)
        cat_scr[:, 2 * EMBED:] = jnp.where(m, h_new, h)
        c_scr[...] = jnp.where(m, c_new, c)

    @pl.when(u == SEQ // 2 - 1)
    def _fin():
        out_ref[...] = cat_scr[:, 2 * EMBED:]


def _tc_lstm(len2, xs, wcat_e, wcat_o, bias, interpret=False):
    return pl.pallas_call(
        _lstm_body,
        grid=(SEQ // 2,),
        in_specs=[
            pl.BlockSpec((BATCH, 1), lambda u: (0, 0)),
            pl.BlockSpec((1, BATCH, 2 * EMBED), lambda u: (u, 0, 0)),
            pl.BlockSpec((2 * EMBED + HIDDEN, 4 * HIDDEN), lambda u: (0, 0)),
            pl.BlockSpec((2 * EMBED + HIDDEN, 4 * HIDDEN), lambda u: (0, 0)),
            pl.BlockSpec((1, 4 * HIDDEN), lambda u: (0, 0)),
        ],
        out_specs=pl.BlockSpec((BATCH, HIDDEN), lambda u: (0, 0)),
        out_shape=jax.ShapeDtypeStruct((BATCH, HIDDEN), jnp.float32),
        scratch_shapes=[
            pltpu.VMEM((BATCH, 2 * EMBED + HIDDEN), jnp.float32),
            pltpu.VMEM((BATCH, HIDDEN), jnp.float32),
        ],
        interpret=interpret,
    )(len2, xs, wcat_e, wcat_o, bias)


def kernel(left, left_len, word_emb, W_ih, W_hh, b_ih, b_hh):
    emb_flat = _sc_gather_pipeline(word_emb, left.astype(jnp.int32))
    xs = emb_flat.reshape(SEQ // 2, BATCH, 2 * EMBED)
    scale = jnp.concatenate(
        [
            jnp.full((2 * HIDDEN,), 0.5, jnp.float32),
            jnp.ones((HIDDEN,), jnp.float32),
            jnp.full((HIDDEN,), 0.5, jnp.float32),
        ]
    )
    wx = W_ih.T * scale[None, :]
    wh = W_hh.T * scale[None, :]
    z = jnp.zeros((EMBED, 4 * HIDDEN), jnp.float32)
    wcat_e = jnp.concatenate([wx, z, wh])
    wcat_o = jnp.concatenate([z, wx, wh])
    bias = ((b_ih + b_hh) * scale).reshape(1, 4 * HIDDEN)
    len2 = left_len.reshape(BATCH, 1).astype(jnp.int32)
    return _tc_lstm(len2, xs, wcat_e, wcat_o, bias)
